# baseline (speedup 1.0000x reference)
import functools

import jax
import jax.numpy as jnp
from jax import lax
from jax.experimental import pallas as pl
from jax.experimental.pallas import tpu as pltpu
from jax.experimental.pallas import tpu_sc as plsc

B = 16384
D = 32
NC = 2
NS = 16
NW = NC * NS
BPW = B // NW
CHUNK = 128
NCHUNK = BPW // CHUNK
GROUPS = BPW // 16


@functools.partial(
    pl.kernel,
    mesh=plsc.VectorSubcoreMesh(core_axis_name="c", subcore_axis_name="s"),
    out_type=jax.ShapeDtypeStruct((B,), jnp.float32),
    compiler_params=pltpu.CompilerParams(
        needs_layout_passes=False, use_tc_tiling_on_sc=False),
    scratch_types=[
        pltpu.VMEM((NCHUNK, CHUNK), jnp.int32),
        pltpu.VMEM((NCHUNK, CHUNK), jnp.int32),
        pltpu.VMEM((BPW, D), jnp.float32),
        pltpu.VMEM((BPW, D), jnp.float32),
        pltpu.VMEM((BPW,), jnp.float32),
        pltpu.SemaphoreType.DMA,
    ],
)
def _mf_kernel(users_hbm, items_hbm, uf_hbm, if_hbm, out_hbm,
               uidx_v, iidx_v, urows_v, vrows_v, out_v, sem):
    wid = lax.axis_index("s") * NC + lax.axis_index("c")
    base = wid * BPW

    pltpu.sync_copy(users_hbm.at[pl.ds(wid * NCHUNK, NCHUNK)], uidx_v)
    pltpu.sync_copy(items_hbm.at[pl.ds(wid * NCHUNK, NCHUNK)], iidx_v)

    copies = []
    for j in range(NCHUNK):
        copies.append(pltpu.async_copy(
            uf_hbm.at[uidx_v.at[j]], urows_v.at[pl.ds(j * CHUNK, CHUNK)], sem))
        copies.append(pltpu.async_copy(
            if_hbm.at[iidx_v.at[j]], vrows_v.at[pl.ds(j * CHUNK, CHUNK)], sem))
    for c in copies:
        c.wait()

    lanes = lax.iota(jnp.int32, 16)

    def group_body(g, carry):
        res = jnp.zeros((16,), jnp.float32)
        for dr in range(16):
            row = g * 16 + dr
            u0 = urows_v[row, pl.ds(0, 16)]
            u1 = urows_v[row, pl.ds(16, 16)]
            v0 = vrows_v[row, pl.ds(0, 16)]
            v1 = vrows_v[row, pl.ds(16, 16)]
            s = u0 * v0 + u1 * v1
            tot = jnp.sum(s)
            res = jnp.where(lanes == dr, tot, res)
        out_v[pl.ds(g * 16, 16)] = res
        return carry

    lax.fori_loop(0, GROUPS, group_body, 0)

    pltpu.sync_copy(out_v, out_hbm.at[pl.ds(base, BPW)])


def kernel(data, user_factors, item_factors):
    users = data[:, 0].reshape(NW * NCHUNK, CHUNK)
    items = data[:, 1].reshape(NW * NCHUNK, CHUNK)
    return _mf_kernel(users, items, user_factors, item_factors)
